# baseline (speedup 1.0000x reference)
import functools

import jax
import jax.numpy as jnp
from jax import lax
from jax.experimental import pallas as pl
from jax.experimental.pallas import tpu as pltpu
from jax.experimental.pallas import tpu_sc as plsc

N = 10000
E = 160000
IN_DIM = 256
HID = 256
HEADS = 4
OUT_DIM = 64
FEAT = HEADS * HID
HPW = FEAT + 128
DNW = 64
RB = 1000
NBLK = N // RB

E2 = E + N
NTILES = 16
TQ = 10640
E2P = TQ * NTILES
NW = 32
CHT = 56
SWEEPS = 6
NPAD = NW * CHT * SWEEPS
ROWS_T = CHT + 2
ADS = CHT + 8
EB = 16
LB = 16
SCCH = 2432
QC = 2112
QCF = 2048


def _tk1_body(x_ref, W_ref, As_ref, Ad_ref, h_ref, ab_ref):
    h = jnp.dot(x_ref[...], W_ref[...], preferred_element_type=jnp.float32)
    h_ref[...] = jnp.concatenate(
        [h, jnp.dot(h, As_ref[...], preferred_element_type=jnp.float32)], axis=1)
    ab_ref[...] = jnp.dot(h, Ad_ref[...], preferred_element_type=jnp.float32)


def _tk2_body(accf_ref, dn_ref, S_ref, b_ref, W_ref, As_ref, Ad_ref, h_ref, ab_ref):
    den = dn_ref[:, :4]
    recip = 1.0 / (den + 1e-16)
    scale = jnp.dot(recip, S_ref[...], preferred_element_type=jnp.float32)
    hn = jnp.maximum(accf_ref[...] * scale + b_ref[...], 0.0)
    h = jnp.dot(hn, W_ref[...], preferred_element_type=jnp.float32)
    h_ref[...] = jnp.concatenate(
        [h, jnp.dot(h, As_ref[...], preferred_element_type=jnp.float32)], axis=1)
    ab_ref[...] = jnp.dot(h, Ad_ref[...], preferred_element_type=jnp.float32)


def _tk3_body(accf_ref, dn_ref, S_ref, b_ref, Wc1_ref, bc1_ref, Wc2_ref, bc2_ref, o_ref):
    den = dn_ref[:, :4]
    recip = 1.0 / (den + 1e-16)
    scale = jnp.dot(recip, S_ref[...], preferred_element_type=jnp.float32)
    hn = jnp.maximum(accf_ref[...] * scale + b_ref[...], 0.0)
    c = jnp.maximum(
        jnp.dot(hn, Wc1_ref[...], preferred_element_type=jnp.float32) + bc1_ref[...],
        0.0)
    o_ref[...] = jnp.dot(c, Wc2_ref[...], preferred_element_type=jnp.float32) + bc2_ref[...]


def _row_spec(w):
    return pl.BlockSpec((RB, w), lambda i: (i, 0))


def _full_spec(shape):
    return pl.BlockSpec(shape, lambda i: tuple(0 for _ in shape))


def _tk1(x, W1, As1, Ad1):
    return pl.pallas_call(
        _tk1_body,
        grid=(NBLK,),
        in_specs=[_row_spec(IN_DIM), _full_spec((IN_DIM, FEAT)),
                  _full_spec((FEAT, 128)), _full_spec((FEAT, 128))],
        out_specs=[_row_spec(HPW), _row_spec(128)],
        out_shape=[jax.ShapeDtypeStruct((N, HPW), jnp.float32),
                   jax.ShapeDtypeStruct((N, 128), jnp.float32)],
    )(x, W1, As1, Ad1)


def _tk2(accf, dn, S, b, W2, As2, Ad2):
    return pl.pallas_call(
        _tk2_body,
        grid=(NBLK,),
        in_specs=[_row_spec(FEAT), _row_spec(DNW), _full_spec((4, FEAT)),
                  _full_spec((1, FEAT)), _full_spec((FEAT, FEAT)),
                  _full_spec((FEAT, 128)), _full_spec((FEAT, 128))],
        out_specs=[_row_spec(HPW), _row_spec(128)],
        out_shape=[jax.ShapeDtypeStruct((N, HPW), jnp.float32),
                   jax.ShapeDtypeStruct((N, 128), jnp.float32)],
    )(accf, dn, S, b, W2, As2, Ad2)


def _tk3(accf, dn, S, b, Wc1, bc1, Wc2, bc2):
    return pl.pallas_call(
        _tk3_body,
        grid=(NBLK,),
        in_specs=[_row_spec(FEAT), _row_spec(DNW), _full_spec((4, FEAT)),
                  _full_spec((1, FEAT)), _full_spec((FEAT, HID)),
                  _full_spec((1, HID)), _full_spec((HID, OUT_DIM)),
                  _full_spec((1, OUT_DIM))],
        out_specs=_row_spec(OUT_DIM),
        out_shape=jax.ShapeDtypeStruct((N, OUT_DIM), jnp.float32),
    )(accf, dn, S, b, Wc1, bc1, Wc2, bc2)


def _sc_body(h_hbm, ab_hbm, src_hbm, dst_hbm, accf_hbm, dn_hbm,
             sch_s, sch_d, qpk, adst_loc, rows, tbuf, idx_src, idx_dl,
             accf_t, dn_t, sem, sem2):
    c = lax.axis_index("c")
    sid = lax.axis_index("s")
    w = sid * 2 + c
    lanes = lax.iota(jnp.int32, LB)
    lanesEB = lanes * EB
    zf16 = jnp.zeros((LB,), jnp.float32)

    for i in range(512 // LB):
        tbuf[pl.ds(i * LB, LB)] = zf16
    for i in range(HEADS * ADS // LB):
        adst_loc[pl.ds(i * LB, LB)] = zf16

    def process_queue(qn):
        pad = jnp.full((LB,), CHT << 16, jnp.int32)
        qpk[pl.ds(qn, LB)] = pad
        qpk[pl.ds(qn + LB, LB)] = pad
        nb = (qn + EB - 1) // EB

        def qissue(j, b):
            qv = qpk[pl.ds(j * EB, EB)]
            idx_src[b, pl.ds(0, EB)] = qv & 0xFFFF
            idx_dl[b, pl.ds(0, EB)] = qv >> 16
            pltpu.async_copy(h_hbm.at[idx_src.at[b]], rows.at[b], sem)
            return 0

        qissue(0, 0)

        def batch(j, _):
            b = j & 1
            pltpu.make_async_copy(h_hbm.at[idx_src.at[b]], rows.at[b], sem).wait()
            _ = lax.cond(j + 1 < nb, lambda: qissue(j + 1, 1 - b), lambda: 0)
            dlv = idx_dl[b, pl.ds(0, EB)]
            for h in range(HEADS):
                av = plsc.load_gather(rows.at[b],
                                      [lanes, jnp.full((LB,), FEAT + h, jnp.int32)])
                bv = plsc.load_gather(adst_loc, [dlv + h * ADS])
                e = av + bv
                e = jnp.where(e >= 0, e, 0.2 * e)
                tbuf[pl.ds(h * EB, LB)] = jnp.exp(e)

            def scale(r, _):
                rsplat = plsc.load_gather(idx_dl.at[b], [jnp.full((LB,), r, jnp.int32)])
                trow = plsc.load_gather(tbuf, [lanesEB + r])
                plsc.addupdate_scatter(dn_t, [rsplat, lanes], trow)
                for h in range(HEADS):
                    tsp = plsc.load_gather(
                        tbuf, [jnp.full((LB,), h * EB, jnp.int32) + r])
                    for k in range(HID // LB):
                        col = h * HID + k * LB
                        v = rows[b, r, pl.ds(col, LB)] * tsp
                        plsc.addupdate_scatter(accf_t, [rsplat, lanes + col], v)
                return 0
            lax.fori_loop(0, EB, scale, 0)
            return 0
        lax.fori_loop(0, nb, batch, 0)
        return jnp.int32(0)

    def sweep(s, _carry):
        base = (s * NW + w) * CHT
        def za(i, _):
            accf_t[i // (FEAT // LB), pl.ds((i % (FEAT // LB)) * LB, LB)] = zf16
            return 0
        lax.fori_loop(0, ROWS_T * FEAT // LB, za, 0)
        def zd(i, _):
            dn_t[i // (DNW // LB), pl.ds((i % (DNW // LB)) * LB, LB)] = zf16
            return 0
        lax.fori_loop(0, ROWS_T * DNW // LB, zd, 0)
        for h in range(HEADS):
            pltpu.sync_copy(ab_hbm.at[pl.ds(h * NPAD + base, CHT)],
                            adst_loc.at[pl.ds(h * ADS, CHT)])


        def issue(ci, b):
            off = ci * SCCH
            pltpu.async_copy(src_hbm.at[pl.ds(off, SCCH)], sch_s.at[b], sem2)
            pltpu.async_copy(dst_hbm.at[pl.ds(off, SCCH)], sch_d.at[b], sem2)
            return 0

        def wait_chunk(ci, b):
            off = ci * SCCH
            pltpu.make_async_copy(src_hbm.at[pl.ds(off, SCCH)], sch_s.at[b], sem2).wait()
            pltpu.make_async_copy(dst_hbm.at[pl.ds(off, SCCH)], sch_d.at[b], sem2).wait()

        issue(0, 0)

        def scan_chunk(ci, qn):
            b = ci & 1
            wait_chunk(ci, b)
            _ = lax.cond(ci + 1 < E2P // SCCH,
                         lambda: issue(ci + 1, 1 - b), lambda: 0)
            def scan(i, qn):
                dvec = sch_d[b, pl.ds(i * LB, LB)]
                m = (dvec >= base) & (dvec < base + CHT)
                cnt = plsc.all_reduce_population_count(m)[0]
                def found(qn):
                    svec = sch_s[b, pl.ds(i * LB, LB)]
                    pos = plsc.cumsum(m.astype(jnp.int32))
                    idx = qn + pos - 1
                    plsc.store_scatter(qpk, [idx],
                                       svec | ((dvec - base) << 16), mask=m)
                    qn = qn + cnt
                    return lax.cond(qn >= QCF, process_queue, lambda q: q, qn)
                return lax.cond(cnt > 0, found, lambda q: q, qn)
            return lax.fori_loop(0, SCCH // LB, scan, qn)
        qn = lax.fori_loop(0, E2P // SCCH, scan_chunk, jnp.int32(0))
        _ = process_queue(qn)

        for i in range(CHT // 8):
            pltpu.sync_copy(accf_t.at[pl.ds(i * 8, 8)],
                            accf_hbm.at[pl.ds(base + i * 8, 8)])
            pltpu.sync_copy(dn_t.at[pl.ds(i * 8, 8)],
                            dn_hbm.at[pl.ds(base + i * 8, 8)])
        return 0

    lax.fori_loop(0, SWEEPS, sweep, 0)


@functools.partial(
    pl.kernel,
    out_type=[jax.ShapeDtypeStruct((NPAD, FEAT), jnp.float32),
              jax.ShapeDtypeStruct((NPAD, DNW), jnp.float32)],
    mesh=plsc.VectorSubcoreMesh(core_axis_name="c", subcore_axis_name="s"),
    compiler_params=pltpu.CompilerParams(needs_layout_passes=False),
    scratch_types=[
        pltpu.VMEM((2, SCCH), jnp.int32),
        pltpu.VMEM((2, SCCH), jnp.int32),
        pltpu.VMEM((QC,), jnp.int32),
        pltpu.VMEM((HEADS * ADS,), jnp.float32),
        pltpu.VMEM((2, EB, HPW), jnp.float32),
        pltpu.VMEM((512,), jnp.float32),
        pltpu.VMEM((2, EB), jnp.int32),
        pltpu.VMEM((2, EB), jnp.int32),
        pltpu.VMEM((ROWS_T, FEAT), jnp.float32),
        pltpu.VMEM((ROWS_T, DNW), jnp.float32),
        pltpu.SemaphoreType.DMA,
        pltpu.SemaphoreType.DMA,
    ],
)
def _sc_aggregate(h_hbm, ab_hbm, src_hbm, dst_hbm, accf_hbm, dn_hbm, *rest):
    _sc_body(h_hbm, ab_hbm, src_hbm, dst_hbm, accf_hbm, dn_hbm, *rest)


def _mk_A(a):
    A = jnp.zeros((HEADS, HID, 128), jnp.float32)
    A = A.at[jnp.arange(HEADS), :, jnp.arange(HEADS)].set(a)
    return A.reshape(FEAT, 128)


def _flat_adst(ab):
    abT = ab[:, :HEADS].T
    abT = jnp.pad(abT, ((0, 0), (0, NPAD - N)))
    return abT.reshape(HEADS * NPAD)


def kernel(x, edge_index, W1, a_src1, a_dst1, b1, W2, a_src2, a_dst2, b2,
           Wc1, bc1, Wc2, bc2):
    loop = jnp.arange(N, dtype=jnp.int32)
    pad = E2P - E2
    src = jnp.concatenate([edge_index[0].astype(jnp.int32), loop,
                           jnp.zeros((pad,), jnp.int32)])
    dst = jnp.concatenate([edge_index[1].astype(jnp.int32), loop,
                           jnp.full((pad,), -1, jnp.int32)])

    As1, Ad1 = _mk_A(a_src1), _mk_A(a_dst1)
    As2, Ad2 = _mk_A(a_src2), _mk_A(a_dst2)
    S = (jnp.arange(FEAT)[None, :] // HID == jnp.arange(HEADS)[:, None]).astype(jnp.float32)

    h1, ab1 = _tk1(x, W1, As1, Ad1)
    accf1, dn1 = _sc_aggregate(h1, _flat_adst(ab1), src, dst)
    h2, ab2 = _tk2(accf1[:N], dn1[:N], S, b1.reshape(1, FEAT), W2, As2, Ad2)
    accf2, dn2 = _sc_aggregate(h2, _flat_adst(ab2), src, dst)
    return _tk3(accf2[:N], dn2[:N], S, b2.reshape(1, FEAT), Wc1,
                bc1.reshape(1, HID), Wc2, bc2.reshape(1, OUT_DIM))


# speedup vs baseline: 1.4499x; 1.4499x over previous
import functools

import jax
import jax.numpy as jnp
from jax import lax
from jax.experimental import pallas as pl
from jax.experimental.pallas import tpu as pltpu
from jax.experimental.pallas import tpu_sc as plsc

N = 10000
E = 160000
IN_DIM = 256
HID = 256
HEADS = 4
OUT_DIM = 64
FEAT = HEADS * HID
HPW = FEAT + 128
DNW = 64
RB = 1000
NBLK = N // RB

E2 = E + N
NTILES = 16
TQ = 10640
E2P = TQ * NTILES
NW = 32
CHT = 64
SWEEPS = 5
NPAD = NW * CHT * SWEEPS
ROWS_T = CHT + 2
ADS = CHT + 8
EB = 16
LB = 16
SCCH = 2432
QC = 4160
QCF = 4096


def _tk1_body(x_ref, W_ref, As_ref, Ad_ref, h_ref, ab_ref):
    h = jnp.dot(x_ref[...], W_ref[...], preferred_element_type=jnp.float32)
    h_ref[...] = jnp.concatenate(
        [h, jnp.dot(h, As_ref[...], preferred_element_type=jnp.float32)], axis=1)
    ab_ref[...] = jnp.dot(h, Ad_ref[...], preferred_element_type=jnp.float32)


def _tk2_body(accf_ref, dn_ref, S_ref, b_ref, W_ref, As_ref, Ad_ref, h_ref, ab_ref):
    den = dn_ref[:, :4]
    recip = 1.0 / (den + 1e-16)
    scale = jnp.dot(recip, S_ref[...], preferred_element_type=jnp.float32)
    hn = jnp.maximum(accf_ref[...] * scale + b_ref[...], 0.0)
    h = jnp.dot(hn, W_ref[...], preferred_element_type=jnp.float32)
    h_ref[...] = jnp.concatenate(
        [h, jnp.dot(h, As_ref[...], preferred_element_type=jnp.float32)], axis=1)
    ab_ref[...] = jnp.dot(h, Ad_ref[...], preferred_element_type=jnp.float32)


def _tk3_body(accf_ref, dn_ref, S_ref, b_ref, Wc1_ref, bc1_ref, Wc2_ref, bc2_ref, o_ref):
    den = dn_ref[:, :4]
    recip = 1.0 / (den + 1e-16)
    scale = jnp.dot(recip, S_ref[...], preferred_element_type=jnp.float32)
    hn = jnp.maximum(accf_ref[...] * scale + b_ref[...], 0.0)
    c = jnp.maximum(
        jnp.dot(hn, Wc1_ref[...], preferred_element_type=jnp.float32) + bc1_ref[...],
        0.0)
    o_ref[...] = jnp.dot(c, Wc2_ref[...], preferred_element_type=jnp.float32) + bc2_ref[...]


def _row_spec(w):
    return pl.BlockSpec((RB, w), lambda i: (i, 0))


def _full_spec(shape):
    return pl.BlockSpec(shape, lambda i: tuple(0 for _ in shape))


def _tk1(x, W1, As1, Ad1):
    return pl.pallas_call(
        _tk1_body,
        grid=(NBLK,),
        in_specs=[_row_spec(IN_DIM), _full_spec((IN_DIM, FEAT)),
                  _full_spec((FEAT, 128)), _full_spec((FEAT, 128))],
        out_specs=[_row_spec(HPW), _row_spec(128)],
        out_shape=[jax.ShapeDtypeStruct((N, HPW), jnp.float32),
                   jax.ShapeDtypeStruct((N, 128), jnp.float32)],
    )(x, W1, As1, Ad1)


def _tk2(accf, dn, S, b, W2, As2, Ad2):
    return pl.pallas_call(
        _tk2_body,
        grid=(NBLK,),
        in_specs=[_row_spec(FEAT), _row_spec(DNW), _full_spec((4, FEAT)),
                  _full_spec((1, FEAT)), _full_spec((FEAT, FEAT)),
                  _full_spec((FEAT, 128)), _full_spec((FEAT, 128))],
        out_specs=[_row_spec(HPW), _row_spec(128)],
        out_shape=[jax.ShapeDtypeStruct((N, HPW), jnp.float32),
                   jax.ShapeDtypeStruct((N, 128), jnp.float32)],
    )(accf, dn, S, b, W2, As2, Ad2)


def _tk3(accf, dn, S, b, Wc1, bc1, Wc2, bc2):
    return pl.pallas_call(
        _tk3_body,
        grid=(NBLK,),
        in_specs=[_row_spec(FEAT), _row_spec(DNW), _full_spec((4, FEAT)),
                  _full_spec((1, FEAT)), _full_spec((FEAT, HID)),
                  _full_spec((1, HID)), _full_spec((HID, OUT_DIM)),
                  _full_spec((1, OUT_DIM))],
        out_specs=_row_spec(OUT_DIM),
        out_shape=jax.ShapeDtypeStruct((N, OUT_DIM), jnp.float32),
    )(accf, dn, S, b, Wc1, bc1, Wc2, bc2)


def _sc_body(h_hbm, ab_hbm, src_hbm, dst_hbm, accf_hbm, dn_hbm,
             sch_s, sch_d, qpk, adst_loc, rows, tbuf, idx_src, idx_dl,
             accf_t, dn_t, sem, sem2):
    c = lax.axis_index("c")
    sid = lax.axis_index("s")
    w = sid * 2 + c
    lanes = lax.iota(jnp.int32, LB)
    lanesEB = lanes * EB
    zf16 = jnp.zeros((LB,), jnp.float32)

    for i in range(512 // LB):
        tbuf[pl.ds(i * LB, LB)] = zf16
    for i in range(HEADS * ADS // LB):
        adst_loc[pl.ds(i * LB, LB)] = zf16

    def process_queue(qn):
        pad = jnp.full((LB,), CHT << 16, jnp.int32)
        qpk[pl.ds(qn, LB)] = pad
        qpk[pl.ds(qn + LB, LB)] = pad
        nb = (qn + EB - 1) // EB

        def batch(j, _):
            off = j * EB
            for g in range(EB // LB):
                qv = qpk[pl.ds(off + g * LB, LB)]
                idx_src[pl.ds(g * LB, LB)] = qv & 0xFFFF
                idx_dl[pl.ds(g * LB, LB)] = qv >> 16
            pltpu.async_copy(h_hbm.at[idx_src], rows, sem).wait()
            for g in range(EB // LB):
                dlv = idx_dl[pl.ds(g * LB, LB)]
                gl = lanes + g * LB
                for h in range(HEADS):
                    av = plsc.load_gather(rows, [gl, jnp.full((LB,), FEAT + h, jnp.int32)])
                    bv = plsc.load_gather(adst_loc, [dlv + h * ADS])
                    e = av + bv
                    e = jnp.where(e >= 0, e, 0.2 * e)
                    tbuf[pl.ds(h * EB + g * LB, LB)] = jnp.exp(e)

            def scale(r, _):
                rsplat = plsc.load_gather(idx_dl, [jnp.full((LB,), r, jnp.int32)])
                trow = plsc.load_gather(tbuf, [lanesEB + r])
                plsc.addupdate_scatter(dn_t, [rsplat, lanes], trow)
                for h in range(HEADS):
                    tsp = plsc.load_gather(
                        tbuf, [jnp.full((LB,), h * EB, jnp.int32) + r])
                    for k in range(HID // LB):
                        col = h * HID + k * LB
                        v = rows[r, pl.ds(col, LB)] * tsp
                        plsc.addupdate_scatter(accf_t, [rsplat, lanes + col], v)
                return 0
            lax.fori_loop(0, EB, scale, 0)
            return 0
        lax.fori_loop(0, nb, batch, 0)
        return jnp.int32(0)

    def sweep(s, _carry):
        base = (s * NW + w) * CHT
        def za(i, _):
            accf_t[i // (FEAT // LB), pl.ds((i % (FEAT // LB)) * LB, LB)] = zf16
            return 0
        lax.fori_loop(0, ROWS_T * FEAT // LB, za, 0)
        def zd(i, _):
            dn_t[i // (DNW // LB), pl.ds((i % (DNW // LB)) * LB, LB)] = zf16
            return 0
        lax.fori_loop(0, ROWS_T * DNW // LB, zd, 0)
        for h in range(HEADS):
            pltpu.sync_copy(ab_hbm.at[pl.ds(h * NPAD + base, CHT)],
                            adst_loc.at[pl.ds(h * ADS, CHT)])


        def issue(ci, b):
            off = ci * SCCH
            pltpu.async_copy(src_hbm.at[pl.ds(off, SCCH)], sch_s.at[b], sem2)
            pltpu.async_copy(dst_hbm.at[pl.ds(off, SCCH)], sch_d.at[b], sem2)
            return 0

        def wait_chunk(ci, b):
            off = ci * SCCH
            pltpu.make_async_copy(src_hbm.at[pl.ds(off, SCCH)], sch_s.at[b], sem2).wait()
            pltpu.make_async_copy(dst_hbm.at[pl.ds(off, SCCH)], sch_d.at[b], sem2).wait()

        issue(0, 0)

        def scan_chunk(ci, qn):
            b = ci & 1
            wait_chunk(ci, b)
            _ = lax.cond(ci + 1 < E2P // SCCH,
                         lambda: issue(ci + 1, 1 - b), lambda: 0)
            def scan(i, qn):
                dvec = sch_d[b, pl.ds(i * LB, LB)]
                m = (dvec >= base) & (dvec < base + CHT)
                cnt = plsc.all_reduce_population_count(m)[0]
                def found(qn):
                    svec = sch_s[b, pl.ds(i * LB, LB)]
                    pos = plsc.cumsum(m.astype(jnp.int32))
                    idx = qn + pos - 1
                    plsc.store_scatter(qpk, [idx],
                                       svec | ((dvec - base) << 16), mask=m)
                    qn = qn + cnt
                    return lax.cond(qn >= QCF, process_queue, lambda q: q, qn)
                return lax.cond(cnt > 0, found, lambda q: q, qn)
            return lax.fori_loop(0, SCCH // LB, scan, qn)
        qn = lax.fori_loop(0, E2P // SCCH, scan_chunk, jnp.int32(0))
        _ = process_queue(qn)

        for i in range(CHT // 8):
            pltpu.sync_copy(accf_t.at[pl.ds(i * 8, 8)],
                            accf_hbm.at[pl.ds(base + i * 8, 8)])
            pltpu.sync_copy(dn_t.at[pl.ds(i * 8, 8)],
                            dn_hbm.at[pl.ds(base + i * 8, 8)])
        return 0

    lax.fori_loop(0, SWEEPS, sweep, 0)


@functools.partial(
    pl.kernel,
    out_type=[jax.ShapeDtypeStruct((NPAD, FEAT), jnp.float32),
              jax.ShapeDtypeStruct((NPAD, DNW), jnp.float32)],
    mesh=plsc.VectorSubcoreMesh(core_axis_name="c", subcore_axis_name="s"),
    compiler_params=pltpu.CompilerParams(needs_layout_passes=False),
    scratch_types=[
        pltpu.VMEM((2, SCCH), jnp.int32),
        pltpu.VMEM((2, SCCH), jnp.int32),
        pltpu.VMEM((QC,), jnp.int32),
        pltpu.VMEM((HEADS * ADS,), jnp.float32),
        pltpu.VMEM((EB, HPW), jnp.float32),
        pltpu.VMEM((512,), jnp.float32),
        pltpu.VMEM((EB,), jnp.int32),
        pltpu.VMEM((EB,), jnp.int32),
        pltpu.VMEM((ROWS_T, FEAT), jnp.float32),
        pltpu.VMEM((ROWS_T, DNW), jnp.float32),
        pltpu.SemaphoreType.DMA,
        pltpu.SemaphoreType.DMA,
    ],
)
def _sc_aggregate(h_hbm, ab_hbm, src_hbm, dst_hbm, accf_hbm, dn_hbm, *rest):
    _sc_body(h_hbm, ab_hbm, src_hbm, dst_hbm, accf_hbm, dn_hbm, *rest)


def _mk_A(a):
    A = jnp.zeros((HEADS, HID, 128), jnp.float32)
    A = A.at[jnp.arange(HEADS), :, jnp.arange(HEADS)].set(a)
    return A.reshape(FEAT, 128)


def _flat_adst(ab):
    abT = ab[:, :HEADS].T
    abT = jnp.pad(abT, ((0, 0), (0, NPAD - N)))
    return abT.reshape(HEADS * NPAD)


def kernel(x, edge_index, W1, a_src1, a_dst1, b1, W2, a_src2, a_dst2, b2,
           Wc1, bc1, Wc2, bc2):
    loop = jnp.arange(N, dtype=jnp.int32)
    pad = E2P - E2
    src = jnp.concatenate([edge_index[0].astype(jnp.int32), loop,
                           jnp.zeros((pad,), jnp.int32)])
    dst = jnp.concatenate([edge_index[1].astype(jnp.int32), loop,
                           jnp.full((pad,), -1, jnp.int32)])

    As1, Ad1 = _mk_A(a_src1), _mk_A(a_dst1)
    As2, Ad2 = _mk_A(a_src2), _mk_A(a_dst2)
    S = (jnp.arange(FEAT)[None, :] // HID == jnp.arange(HEADS)[:, None]).astype(jnp.float32)

    h1, ab1 = _tk1(x, W1, As1, Ad1)
    accf1, dn1 = _sc_aggregate(h1, _flat_adst(ab1), src, dst)
    h2, ab2 = _tk2(accf1[:N], dn1[:N], S, b1.reshape(1, FEAT), W2, As2, Ad2)
    accf2, dn2 = _sc_aggregate(h2, _flat_adst(ab2), src, dst)
    return _tk3(accf2[:N], dn2[:N], S, b2.reshape(1, FEAT), Wc1,
                bc1.reshape(1, HID), Wc2, bc2.reshape(1, OUT_DIM))


# speedup vs baseline: 1.8054x; 1.2452x over previous
import functools

import jax
import jax.numpy as jnp
from jax import lax
from jax.experimental import pallas as pl
from jax.experimental.pallas import tpu as pltpu
from jax.experimental.pallas import tpu_sc as plsc

N = 10000
E = 160000
IN_DIM = 256
HID = 256
HEADS = 4
OUT_DIM = 64
FEAT = HEADS * HID
HPW = FEAT + 128
DNW = 64
RB = 1000
NBLK = N // RB

E2 = E + N
NTILES = 16
TQ = 10640
E2P = TQ * NTILES
NW = 32
CHT = 64
SWEEPS = 5
NPAD = NW * CHT * SWEEPS
ROWS_T = CHT + 2
ADS = CHT + 8
EB = 16
LB = 16
SCCH = 2432
QC = 4160
QCF = 4096


def _tk1_body(x_ref, W_ref, As_ref, Ad_ref, h_ref, ab_ref):
    h = jnp.dot(x_ref[...], W_ref[...], preferred_element_type=jnp.float32)
    h_ref[...] = jnp.concatenate(
        [h, jnp.dot(h, As_ref[...], preferred_element_type=jnp.float32)], axis=1)
    ab_ref[...] = jnp.dot(h, Ad_ref[...], preferred_element_type=jnp.float32)


def _tk2_body(accf_ref, dn_ref, S_ref, b_ref, W_ref, As_ref, Ad_ref, h_ref, ab_ref):
    den = dn_ref[:, :4]
    recip = 1.0 / (den + 1e-16)
    scale = jnp.dot(recip, S_ref[...], preferred_element_type=jnp.float32)
    hn = jnp.maximum(accf_ref[...] * scale + b_ref[...], 0.0)
    h = jnp.dot(hn, W_ref[...], preferred_element_type=jnp.float32)
    h_ref[...] = jnp.concatenate(
        [h, jnp.dot(h, As_ref[...], preferred_element_type=jnp.float32)], axis=1)
    ab_ref[...] = jnp.dot(h, Ad_ref[...], preferred_element_type=jnp.float32)


def _tk3_body(accf_ref, dn_ref, S_ref, b_ref, Wc1_ref, bc1_ref, Wc2_ref, bc2_ref, o_ref):
    den = dn_ref[:, :4]
    recip = 1.0 / (den + 1e-16)
    scale = jnp.dot(recip, S_ref[...], preferred_element_type=jnp.float32)
    hn = jnp.maximum(accf_ref[...] * scale + b_ref[...], 0.0)
    c = jnp.maximum(
        jnp.dot(hn, Wc1_ref[...], preferred_element_type=jnp.float32) + bc1_ref[...],
        0.0)
    o_ref[...] = jnp.dot(c, Wc2_ref[...], preferred_element_type=jnp.float32) + bc2_ref[...]


def _row_spec(w):
    return pl.BlockSpec((RB, w), lambda i: (i, 0))


def _full_spec(shape):
    return pl.BlockSpec(shape, lambda i: tuple(0 for _ in shape))


def _tk1(x, W1, As1, Ad1):
    return pl.pallas_call(
        _tk1_body,
        grid=(NBLK,),
        in_specs=[_row_spec(IN_DIM), _full_spec((IN_DIM, FEAT)),
                  _full_spec((FEAT, 128)), _full_spec((FEAT, 128))],
        out_specs=[_row_spec(HPW), _row_spec(128)],
        out_shape=[jax.ShapeDtypeStruct((N, HPW), jnp.float32),
                   jax.ShapeDtypeStruct((N, 128), jnp.float32)],
    )(x, W1, As1, Ad1)


def _tk2(accf, dn, S, b, W2, As2, Ad2):
    return pl.pallas_call(
        _tk2_body,
        grid=(NBLK,),
        in_specs=[_row_spec(FEAT), _row_spec(DNW), _full_spec((4, FEAT)),
                  _full_spec((1, FEAT)), _full_spec((FEAT, FEAT)),
                  _full_spec((FEAT, 128)), _full_spec((FEAT, 128))],
        out_specs=[_row_spec(HPW), _row_spec(128)],
        out_shape=[jax.ShapeDtypeStruct((N, HPW), jnp.float32),
                   jax.ShapeDtypeStruct((N, 128), jnp.float32)],
    )(accf, dn, S, b, W2, As2, Ad2)


def _tk3(accf, dn, S, b, Wc1, bc1, Wc2, bc2):
    return pl.pallas_call(
        _tk3_body,
        grid=(NBLK,),
        in_specs=[_row_spec(FEAT), _row_spec(DNW), _full_spec((4, FEAT)),
                  _full_spec((1, FEAT)), _full_spec((FEAT, HID)),
                  _full_spec((1, HID)), _full_spec((HID, OUT_DIM)),
                  _full_spec((1, OUT_DIM))],
        out_specs=_row_spec(OUT_DIM),
        out_shape=jax.ShapeDtypeStruct((N, OUT_DIM), jnp.float32),
    )(accf, dn, S, b, Wc1, bc1, Wc2, bc2)


def _sc_body(h_hbm, ab_hbm, src_hbm, dst_hbm, accf_hbm, dn_hbm,
             sch_s, sch_d, qpk, adst_loc, rows, tbuf, idx_src, idx_dl,
             accf_t, dn_t, sem, sem2):
    c = lax.axis_index("c")
    sid = lax.axis_index("s")
    w = sid * 2 + c
    lanes = lax.iota(jnp.int32, LB)
    lanesEB = lanes * EB
    zf16 = jnp.zeros((LB,), jnp.float32)

    for i in range(512 // LB):
        tbuf[pl.ds(i * LB, LB)] = zf16
    for i in range(HEADS * ADS // LB):
        adst_loc[pl.ds(i * LB, LB)] = zf16

    def process_queue(qn):
        pad = jnp.full((LB,), CHT << 16, jnp.int32)
        qpk[pl.ds(qn, LB)] = pad
        qpk[pl.ds(qn + LB, LB)] = pad
        nb = (qn + EB - 1) // EB

        def batch(j, _):
            off = j * EB
            for g in range(EB // LB):
                qv = qpk[pl.ds(off + g * LB, LB)]
                idx_src[pl.ds(g * LB, LB)] = qv & 0xFFFF
                idx_dl[pl.ds(g * LB, LB)] = qv >> 16
            pltpu.async_copy(h_hbm.at[idx_src], rows, sem).wait()
            for g in range(EB // LB):
                dlv = idx_dl[pl.ds(g * LB, LB)]
                gl = lanes + g * LB
                for h in range(HEADS):
                    av = plsc.load_gather(rows, [gl, jnp.full((LB,), FEAT + h, jnp.int32)])
                    bv = plsc.load_gather(adst_loc, [dlv + h * ADS])
                    e = av + bv
                    e = jnp.where(e >= 0, e, 0.2 * e)
                    tbuf[pl.ds(h * EB + g * LB, LB)] = jnp.exp(e)

            @plsc.parallel_loop(0, EB, 1, unroll=2)
            def _scale(r):
                rsplat = plsc.load_gather(idx_dl, [jnp.full((LB,), r, jnp.int32)])
                trow = plsc.load_gather(tbuf, [lanesEB + r])
                plsc.addupdate_scatter(dn_t, [rsplat, lanes], trow)
                for h in range(HEADS):
                    tsp = plsc.load_gather(
                        tbuf, [jnp.full((LB,), h * EB, jnp.int32) + r])
                    for k in range(HID // LB):
                        col = h * HID + k * LB
                        v = rows[r, pl.ds(col, LB)] * tsp
                        plsc.addupdate_scatter(accf_t, [rsplat, lanes + col], v)
            return 0
        lax.fori_loop(0, nb, batch, 0)
        return jnp.int32(0)

    def sweep(s, _carry):
        base = (s * NW + w) * CHT
        def za(i, _):
            accf_t[i // (FEAT // LB), pl.ds((i % (FEAT // LB)) * LB, LB)] = zf16
            return 0
        lax.fori_loop(0, ROWS_T * FEAT // LB, za, 0)
        def zd(i, _):
            dn_t[i // (DNW // LB), pl.ds((i % (DNW // LB)) * LB, LB)] = zf16
            return 0
        lax.fori_loop(0, ROWS_T * DNW // LB, zd, 0)
        for h in range(HEADS):
            pltpu.sync_copy(ab_hbm.at[pl.ds(h * NPAD + base, CHT)],
                            adst_loc.at[pl.ds(h * ADS, CHT)])


        def issue(ci, b):
            off = ci * SCCH
            pltpu.async_copy(src_hbm.at[pl.ds(off, SCCH)], sch_s.at[b], sem2)
            pltpu.async_copy(dst_hbm.at[pl.ds(off, SCCH)], sch_d.at[b], sem2)
            return 0

        def wait_chunk(ci, b):
            off = ci * SCCH
            pltpu.make_async_copy(src_hbm.at[pl.ds(off, SCCH)], sch_s.at[b], sem2).wait()
            pltpu.make_async_copy(dst_hbm.at[pl.ds(off, SCCH)], sch_d.at[b], sem2).wait()

        issue(0, 0)

        def scan_chunk(ci, qn):
            b = ci & 1
            wait_chunk(ci, b)
            _ = lax.cond(ci + 1 < E2P // SCCH,
                         lambda: issue(ci + 1, 1 - b), lambda: 0)
            def scan(i, qn):
                dvec = sch_d[b, pl.ds(i * LB, LB)]
                m = (dvec >= base) & (dvec < base + CHT)
                cnt = plsc.all_reduce_population_count(m)[0]
                def found(qn):
                    svec = sch_s[b, pl.ds(i * LB, LB)]
                    pos = plsc.cumsum(m.astype(jnp.int32))
                    idx = qn + pos - 1
                    plsc.store_scatter(qpk, [idx],
                                       svec | ((dvec - base) << 16), mask=m)
                    qn = qn + cnt
                    return lax.cond(qn >= QCF, process_queue, lambda q: q, qn)
                return lax.cond(cnt > 0, found, lambda q: q, qn)
            return lax.fori_loop(0, SCCH // LB, scan, qn)
        qn = lax.fori_loop(0, E2P // SCCH, scan_chunk, jnp.int32(0))
        _ = process_queue(qn)

        for i in range(CHT // 8):
            pltpu.sync_copy(accf_t.at[pl.ds(i * 8, 8)],
                            accf_hbm.at[pl.ds(base + i * 8, 8)])
            pltpu.sync_copy(dn_t.at[pl.ds(i * 8, 8)],
                            dn_hbm.at[pl.ds(base + i * 8, 8)])
        return 0

    lax.fori_loop(0, SWEEPS, sweep, 0)


@functools.partial(
    pl.kernel,
    out_type=[jax.ShapeDtypeStruct((NPAD, FEAT), jnp.float32),
              jax.ShapeDtypeStruct((NPAD, DNW), jnp.float32)],
    mesh=plsc.VectorSubcoreMesh(core_axis_name="c", subcore_axis_name="s"),
    compiler_params=pltpu.CompilerParams(needs_layout_passes=False),
    scratch_types=[
        pltpu.VMEM((2, SCCH), jnp.int32),
        pltpu.VMEM((2, SCCH), jnp.int32),
        pltpu.VMEM((QC,), jnp.int32),
        pltpu.VMEM((HEADS * ADS,), jnp.float32),
        pltpu.VMEM((EB, HPW), jnp.float32),
        pltpu.VMEM((512,), jnp.float32),
        pltpu.VMEM((EB,), jnp.int32),
        pltpu.VMEM((EB,), jnp.int32),
        pltpu.VMEM((ROWS_T, FEAT), jnp.float32),
        pltpu.VMEM((ROWS_T, DNW), jnp.float32),
        pltpu.SemaphoreType.DMA,
        pltpu.SemaphoreType.DMA,
    ],
)
def _sc_aggregate(h_hbm, ab_hbm, src_hbm, dst_hbm, accf_hbm, dn_hbm, *rest):
    _sc_body(h_hbm, ab_hbm, src_hbm, dst_hbm, accf_hbm, dn_hbm, *rest)


def _mk_A(a):
    A = jnp.zeros((HEADS, HID, 128), jnp.float32)
    A = A.at[jnp.arange(HEADS), :, jnp.arange(HEADS)].set(a)
    return A.reshape(FEAT, 128)


def _flat_adst(ab):
    abT = ab[:, :HEADS].T
    abT = jnp.pad(abT, ((0, 0), (0, NPAD - N)))
    return abT.reshape(HEADS * NPAD)


def kernel(x, edge_index, W1, a_src1, a_dst1, b1, W2, a_src2, a_dst2, b2,
           Wc1, bc1, Wc2, bc2):
    loop = jnp.arange(N, dtype=jnp.int32)
    pad = E2P - E2
    src = jnp.concatenate([edge_index[0].astype(jnp.int32), loop,
                           jnp.zeros((pad,), jnp.int32)])
    dst = jnp.concatenate([edge_index[1].astype(jnp.int32), loop,
                           jnp.full((pad,), -1, jnp.int32)])

    As1, Ad1 = _mk_A(a_src1), _mk_A(a_dst1)
    As2, Ad2 = _mk_A(a_src2), _mk_A(a_dst2)
    S = (jnp.arange(FEAT)[None, :] // HID == jnp.arange(HEADS)[:, None]).astype(jnp.float32)

    h1, ab1 = _tk1(x, W1, As1, Ad1)
    accf1, dn1 = _sc_aggregate(h1, _flat_adst(ab1), src, dst)
    h2, ab2 = _tk2(accf1[:N], dn1[:N], S, b1.reshape(1, FEAT), W2, As2, Ad2)
    accf2, dn2 = _sc_aggregate(h2, _flat_adst(ab2), src, dst)
    return _tk3(accf2[:N], dn2[:N], S, b2.reshape(1, FEAT), Wc1,
                bc1.reshape(1, HID), Wc2, bc2.reshape(1, OUT_DIM))


# speedup vs baseline: 1.9589x; 1.0851x over previous
import functools

import jax
import jax.numpy as jnp
from jax import lax
from jax.experimental import pallas as pl
from jax.experimental.pallas import tpu as pltpu
from jax.experimental.pallas import tpu_sc as plsc

N = 10000
E = 160000
IN_DIM = 256
HID = 256
HEADS = 4
OUT_DIM = 64
FEAT = HEADS * HID
HPW = FEAT + 128
DNW = 64
RB = 1000
NBLK = N // RB

E2 = E + N
NTILES = 16
TQ = 10640
E2P = TQ * NTILES
NW = 32
CHT = 64
SWEEPS = 5
NPAD = NW * CHT * SWEEPS
ROWS_T = CHT + 2
ADS = CHT + 8
EB = 16
LB = 16
SCCH = 2432
QC = 4160
QCF = 4096


def _tk1_body(x_ref, W_ref, As_ref, Ad_ref, h_ref, ab_ref):
    h = jnp.dot(x_ref[...], W_ref[...], preferred_element_type=jnp.float32)
    h_ref[...] = jnp.concatenate(
        [h, jnp.dot(h, As_ref[...], preferred_element_type=jnp.float32)], axis=1)
    ab_ref[...] = jnp.dot(h, Ad_ref[...], preferred_element_type=jnp.float32)


def _tk2_body(accf_ref, dn_ref, S_ref, b_ref, W_ref, As_ref, Ad_ref, h_ref, ab_ref):
    den = dn_ref[:, :4]
    recip = 1.0 / (den + 1e-16)
    scale = jnp.dot(recip, S_ref[...], preferred_element_type=jnp.float32)
    hn = jnp.maximum(accf_ref[...] * scale + b_ref[...], 0.0)
    h = jnp.dot(hn, W_ref[...], preferred_element_type=jnp.float32)
    h_ref[...] = jnp.concatenate(
        [h, jnp.dot(h, As_ref[...], preferred_element_type=jnp.float32)], axis=1)
    ab_ref[...] = jnp.dot(h, Ad_ref[...], preferred_element_type=jnp.float32)


def _tk3_body(accf_ref, dn_ref, S_ref, b_ref, Wc1_ref, bc1_ref, Wc2_ref, bc2_ref, o_ref):
    den = dn_ref[:, :4]
    recip = 1.0 / (den + 1e-16)
    scale = jnp.dot(recip, S_ref[...], preferred_element_type=jnp.float32)
    hn = jnp.maximum(accf_ref[...] * scale + b_ref[...], 0.0)
    c = jnp.maximum(
        jnp.dot(hn, Wc1_ref[...], preferred_element_type=jnp.float32) + bc1_ref[...],
        0.0)
    o_ref[...] = jnp.dot(c, Wc2_ref[...], preferred_element_type=jnp.float32) + bc2_ref[...]


def _row_spec(w):
    return pl.BlockSpec((RB, w), lambda i: (i, 0))


def _full_spec(shape):
    return pl.BlockSpec(shape, lambda i: tuple(0 for _ in shape))


def _tk1(x, W1, As1, Ad1):
    return pl.pallas_call(
        _tk1_body,
        grid=(NBLK,),
        in_specs=[_row_spec(IN_DIM), _full_spec((IN_DIM, FEAT)),
                  _full_spec((FEAT, 128)), _full_spec((FEAT, 128))],
        out_specs=[_row_spec(HPW), _row_spec(128)],
        out_shape=[jax.ShapeDtypeStruct((N, HPW), jnp.float32),
                   jax.ShapeDtypeStruct((N, 128), jnp.float32)],
    )(x, W1, As1, Ad1)


def _tk2(accf, dn, S, b, W2, As2, Ad2):
    return pl.pallas_call(
        _tk2_body,
        grid=(NBLK,),
        in_specs=[_row_spec(FEAT), _row_spec(DNW), _full_spec((4, FEAT)),
                  _full_spec((1, FEAT)), _full_spec((FEAT, FEAT)),
                  _full_spec((FEAT, 128)), _full_spec((FEAT, 128))],
        out_specs=[_row_spec(HPW), _row_spec(128)],
        out_shape=[jax.ShapeDtypeStruct((N, HPW), jnp.float32),
                   jax.ShapeDtypeStruct((N, 128), jnp.float32)],
    )(accf, dn, S, b, W2, As2, Ad2)


def _tk3(accf, dn, S, b, Wc1, bc1, Wc2, bc2):
    return pl.pallas_call(
        _tk3_body,
        grid=(NBLK,),
        in_specs=[_row_spec(FEAT), _row_spec(DNW), _full_spec((4, FEAT)),
                  _full_spec((1, FEAT)), _full_spec((FEAT, HID)),
                  _full_spec((1, HID)), _full_spec((HID, OUT_DIM)),
                  _full_spec((1, OUT_DIM))],
        out_specs=_row_spec(OUT_DIM),
        out_shape=jax.ShapeDtypeStruct((N, OUT_DIM), jnp.float32),
    )(accf, dn, S, b, Wc1, bc1, Wc2, bc2)


def _sc_body(h_hbm, ab_hbm, src_hbm, dst_hbm, accf_hbm, dn_hbm,
             sch_s, sch_d, qpk, adst_loc, rows, tbuf, idx_src, idx_dl,
             accf_t, dn_t, sem, sem2):
    c = lax.axis_index("c")
    sid = lax.axis_index("s")
    w = sid * 2 + c
    lanes = lax.iota(jnp.int32, LB)
    lanesEB = lanes * EB
    zf16 = jnp.zeros((LB,), jnp.float32)

    for i in range(512 // LB):
        tbuf[pl.ds(i * LB, LB)] = zf16
    for i in range(HEADS * ADS // LB):
        adst_loc[pl.ds(i * LB, LB)] = zf16

    def process_queue(qn):
        pad = jnp.full((LB,), CHT << 16, jnp.int32)
        qpk[pl.ds(qn, LB)] = pad
        qpk[pl.ds(qn + LB, LB)] = pad
        nb = (qn + EB - 1) // EB

        def batch(j, _):
            off = j * EB
            for g in range(EB // LB):
                qv = qpk[pl.ds(off + g * LB, LB)]
                idx_src[pl.ds(g * LB, LB)] = qv & 0xFFFF
                idx_dl[pl.ds(g * LB, LB)] = qv >> 16
            pltpu.async_copy(h_hbm.at[idx_src], rows, sem).wait()
            for g in range(EB // LB):
                dlv = idx_dl[pl.ds(g * LB, LB)]
                gl = lanes + g * LB
                for h in range(HEADS):
                    av = plsc.load_gather(rows, [gl, jnp.full((LB,), FEAT + h, jnp.int32)])
                    bv = plsc.load_gather(adst_loc, [dlv + h * ADS])
                    e = av + bv
                    e = jnp.where(e >= 0, e, 0.2 * e)
                    tbuf[pl.ds(h * EB + g * LB, LB)] = jnp.exp(e)

            @plsc.parallel_loop(0, EB, 1, unroll=4)
            def _scale(r):
                rsplat = plsc.load_gather(idx_dl, [jnp.full((LB,), r, jnp.int32)])
                trow = plsc.load_gather(tbuf, [lanesEB + r])
                plsc.addupdate_scatter(dn_t, [rsplat, lanes], trow)
                for h in range(HEADS):
                    tsp = plsc.load_gather(
                        tbuf, [jnp.full((LB,), h * EB, jnp.int32) + r])
                    for k in range(HID // LB):
                        col = h * HID + k * LB
                        v = rows[r, pl.ds(col, LB)] * tsp
                        plsc.addupdate_scatter(accf_t, [rsplat, lanes + col], v)
            return 0
        lax.fori_loop(0, nb, batch, 0)
        return jnp.int32(0)

    def sweep(s, _carry):
        base = (s * NW + w) * CHT
        def za(i, _):
            accf_t[i // (FEAT // LB), pl.ds((i % (FEAT // LB)) * LB, LB)] = zf16
            return 0
        lax.fori_loop(0, ROWS_T * FEAT // LB, za, 0)
        def zd(i, _):
            dn_t[i // (DNW // LB), pl.ds((i % (DNW // LB)) * LB, LB)] = zf16
            return 0
        lax.fori_loop(0, ROWS_T * DNW // LB, zd, 0)
        for h in range(HEADS):
            pltpu.sync_copy(ab_hbm.at[pl.ds(h * NPAD + base, CHT)],
                            adst_loc.at[pl.ds(h * ADS, CHT)])


        def issue(ci, b):
            off = ci * SCCH
            pltpu.async_copy(src_hbm.at[pl.ds(off, SCCH)], sch_s.at[b], sem2)
            pltpu.async_copy(dst_hbm.at[pl.ds(off, SCCH)], sch_d.at[b], sem2)
            return 0

        def wait_chunk(ci, b):
            off = ci * SCCH
            pltpu.make_async_copy(src_hbm.at[pl.ds(off, SCCH)], sch_s.at[b], sem2).wait()
            pltpu.make_async_copy(dst_hbm.at[pl.ds(off, SCCH)], sch_d.at[b], sem2).wait()

        issue(0, 0)

        def scan_chunk(ci, qn):
            b = ci & 1
            wait_chunk(ci, b)
            _ = lax.cond(ci + 1 < E2P // SCCH,
                         lambda: issue(ci + 1, 1 - b), lambda: 0)
            def scan(i, qn):
                dvec = sch_d[b, pl.ds(i * LB, LB)]
                m = (dvec >= base) & (dvec < base + CHT)
                cnt = plsc.all_reduce_population_count(m)[0]
                def found(qn):
                    svec = sch_s[b, pl.ds(i * LB, LB)]
                    pos = plsc.cumsum(m.astype(jnp.int32))
                    idx = qn + pos - 1
                    plsc.store_scatter(qpk, [idx],
                                       svec | ((dvec - base) << 16), mask=m)
                    qn = qn + cnt
                    return lax.cond(qn >= QCF, process_queue, lambda q: q, qn)
                return lax.cond(cnt > 0, found, lambda q: q, qn)
            return lax.fori_loop(0, SCCH // LB, scan, qn)
        qn = lax.fori_loop(0, E2P // SCCH, scan_chunk, jnp.int32(0))
        _ = process_queue(qn)

        for i in range(CHT // 8):
            pltpu.sync_copy(accf_t.at[pl.ds(i * 8, 8)],
                            accf_hbm.at[pl.ds(base + i * 8, 8)])
            pltpu.sync_copy(dn_t.at[pl.ds(i * 8, 8)],
                            dn_hbm.at[pl.ds(base + i * 8, 8)])
        return 0

    lax.fori_loop(0, SWEEPS, sweep, 0)


@functools.partial(
    pl.kernel,
    out_type=[jax.ShapeDtypeStruct((NPAD, FEAT), jnp.float32),
              jax.ShapeDtypeStruct((NPAD, DNW), jnp.float32)],
    mesh=plsc.VectorSubcoreMesh(core_axis_name="c", subcore_axis_name="s"),
    compiler_params=pltpu.CompilerParams(needs_layout_passes=False),
    scratch_types=[
        pltpu.VMEM((2, SCCH), jnp.int32),
        pltpu.VMEM((2, SCCH), jnp.int32),
        pltpu.VMEM((QC,), jnp.int32),
        pltpu.VMEM((HEADS * ADS,), jnp.float32),
        pltpu.VMEM((EB, HPW), jnp.float32),
        pltpu.VMEM((512,), jnp.float32),
        pltpu.VMEM((EB,), jnp.int32),
        pltpu.VMEM((EB,), jnp.int32),
        pltpu.VMEM((ROWS_T, FEAT), jnp.float32),
        pltpu.VMEM((ROWS_T, DNW), jnp.float32),
        pltpu.SemaphoreType.DMA,
        pltpu.SemaphoreType.DMA,
    ],
)
def _sc_aggregate(h_hbm, ab_hbm, src_hbm, dst_hbm, accf_hbm, dn_hbm, *rest):
    _sc_body(h_hbm, ab_hbm, src_hbm, dst_hbm, accf_hbm, dn_hbm, *rest)


def _mk_A(a):
    A = jnp.zeros((HEADS, HID, 128), jnp.float32)
    A = A.at[jnp.arange(HEADS), :, jnp.arange(HEADS)].set(a)
    return A.reshape(FEAT, 128)


def _flat_adst(ab):
    abT = ab[:, :HEADS].T
    abT = jnp.pad(abT, ((0, 0), (0, NPAD - N)))
    return abT.reshape(HEADS * NPAD)


def kernel(x, edge_index, W1, a_src1, a_dst1, b1, W2, a_src2, a_dst2, b2,
           Wc1, bc1, Wc2, bc2):
    loop = jnp.arange(N, dtype=jnp.int32)
    pad = E2P - E2
    src = jnp.concatenate([edge_index[0].astype(jnp.int32), loop,
                           jnp.zeros((pad,), jnp.int32)])
    dst = jnp.concatenate([edge_index[1].astype(jnp.int32), loop,
                           jnp.full((pad,), -1, jnp.int32)])

    As1, Ad1 = _mk_A(a_src1), _mk_A(a_dst1)
    As2, Ad2 = _mk_A(a_src2), _mk_A(a_dst2)
    S = (jnp.arange(FEAT)[None, :] // HID == jnp.arange(HEADS)[:, None]).astype(jnp.float32)

    h1, ab1 = _tk1(x, W1, As1, Ad1)
    accf1, dn1 = _sc_aggregate(h1, _flat_adst(ab1), src, dst)
    h2, ab2 = _tk2(accf1[:N], dn1[:N], S, b1.reshape(1, FEAT), W2, As2, Ad2)
    accf2, dn2 = _sc_aggregate(h2, _flat_adst(ab2), src, dst)
    return _tk3(accf2[:N], dn2[:N], S, b2.reshape(1, FEAT), Wc1,
                bc1.reshape(1, HID), Wc2, bc2.reshape(1, OUT_DIM))
